# baseline (speedup 1.0000x reference)
import functools

import jax
import jax.numpy as jnp
from jax import lax
from jax.experimental import pallas as pl
from jax.experimental.pallas import tpu as pltpu
from jax.experimental.pallas import tpu_sc as plsc

B = 16384
D = 64
W = 128
NW = 32
BPW = B // NW
C = 128
NC = BPW // C


def _sc_kernel(idx_hbm, table_hbm, out_hbm, idx_v, rows_v, gs, ws):
    wid = lax.axis_index("s") * 2 + lax.axis_index("c")
    base = wid * BPW

    pltpu.sync_copy(idx_hbm.at[pl.ds(base, BPW)], idx_v)

    gathers = []
    for j in range(NC):
        gathers.append(pltpu.async_copy(
            table_hbm.at[idx_v.at[pl.ds(j * C, C)]],
            rows_v.at[pl.ds(j * C, C)], gs[j]))
    writes = []
    for j in range(NC):
        gathers[j].wait()
        writes.append(pltpu.async_copy(
            rows_v.at[pl.ds(j * C, C)],
            out_hbm.at[pl.ds(base + j * C, C)], ws[j]))
    for wdma in writes:
        wdma.wait()


@jax.jit
def _run(idx, tablep):
    mesh = plsc.VectorSubcoreMesh(core_axis_name="c", subcore_axis_name="s")
    f = functools.partial(
        pl.kernel, mesh=mesh,
        compiler_params=pltpu.CompilerParams(use_tc_tiling_on_sc=True),
        out_type=jax.ShapeDtypeStruct((B, W), jnp.float32),
        scratch_types=[
            pltpu.VMEM((BPW,), jnp.int32),
            pltpu.VMEM((BPW, W), jnp.float32),
            [pltpu.SemaphoreType.DMA] * NC,
            [pltpu.SemaphoreType.DMA] * NC,
        ],
    )(_sc_kernel)
    return f(idx, tablep)


def kernel(gsp_y_osgb_fourier, gsp_x_osgb_fourier, gsp_id,
           gsp_5_min_time_utc_fourier, emb_table):
    eye = jnp.eye(D, W, dtype=jnp.float32)
    tablep = jax.lax.dot(emb_table, eye,
                         precision=jax.lax.Precision.HIGHEST)
    rows = _run(gsp_id.astype(jnp.int32), tablep)
    out = jnp.concatenate(
        [gsp_y_osgb_fourier[:, 0], gsp_x_osgb_fourier[:, 0],
         rows[:, :D], gsp_5_min_time_utc_fourier], axis=1)
    return out[:, None, :]


# speedup vs baseline: 1.0003x; 1.0003x over previous
import functools

import jax
import jax.numpy as jnp
from jax import lax
from jax.experimental import pallas as pl
from jax.experimental.pallas import tpu as pltpu
from jax.experimental.pallas import tpu_sc as plsc

B = 16384
D = 64
NUM_ROWS = 100000
W = 128
NW = 32
BPW = B // NW
C = 128
NC = BPW // C


def _sc_kernel(idx_hbm, table_hbm, out_hbm, idx_v, rows_v, gs, ws):
    wid = lax.axis_index("s") * 2 + lax.axis_index("c")
    base = wid * BPW

    pltpu.sync_copy(idx_hbm.at[pl.ds(base, BPW)], idx_v)

    gathers = []
    for j in range(NC):
        gathers.append(pltpu.async_copy(
            table_hbm.at[idx_v.at[pl.ds(j * C, C)]],
            rows_v.at[pl.ds(j * C, C)], gs[j]))
    writes = []
    for j in range(NC):
        gathers[j].wait()
        writes.append(pltpu.async_copy(
            rows_v.at[pl.ds(j * C, C)],
            out_hbm.at[pl.ds(base + j * C, C)], ws[j]))
    for wdma in writes:
        wdma.wait()


@jax.jit
def _run(idx, tablep):
    mesh = plsc.VectorSubcoreMesh(core_axis_name="c", subcore_axis_name="s")
    f = functools.partial(
        pl.kernel, mesh=mesh,
        compiler_params=pltpu.CompilerParams(use_tc_tiling_on_sc=True),
        out_type=jax.ShapeDtypeStruct((B, W), jnp.float32),
        scratch_types=[
            pltpu.VMEM((BPW,), jnp.int32),
            pltpu.VMEM((BPW, W), jnp.float32),
            [pltpu.SemaphoreType.DMA] * NC,
            [pltpu.SemaphoreType.DMA] * NC,
        ],
    )(_sc_kernel)
    return f(idx, tablep)


def kernel(gsp_y_osgb_fourier, gsp_x_osgb_fourier, gsp_id,
           gsp_5_min_time_utc_fourier, emb_table):
    eye = jnp.eye(D, W, dtype=jnp.float32)
    tablep = jax.lax.dot(emb_table, eye,
                         precision=jax.lax.Precision.HIGHEST)
    rows = _run(gsp_id.astype(jnp.int32), tablep)
    out = jnp.concatenate(
        [gsp_y_osgb_fourier[:, 0], gsp_x_osgb_fourier[:, 0],
         rows[:, :D], gsp_5_min_time_utc_fourier], axis=1)
    return out[:, None, :]
